# baseline (speedup 1.0000x reference)
import jax
import jax.numpy as jnp
from jax import lax
from jax.experimental import pallas as pl
from jax.experimental.pallas import tpu as pltpu
from jax.experimental.pallas import tpu_sc as plsc

N = 10000
E = 320000
D_IN = 128
D_H = 128
D_OUT = 40
D2P = 64
D2G = 128
BN_EPS = 1e-5

NC = 2
NS = 16
NW = NC * NS
EPW = E // NW
CH = 80
NCHUNK = EPW // CH
NTRIPLE = 41
RINIT = 640
NFULL = 8
NLAST = 5
TAIL0 = 15 * RINIT
TAILN = N - TAIL0

_MESH = plsc.VectorSubcoreMesh(core_axis_name="c", subcore_axis_name="s")


def _ring_edge_loop(x_h, src_h, dst_h, base, rows, isx, idx, isem, gsem,
                    scatter, drain):
    def start_idx(c, k):
        off = base + c * CH
        pltpu.async_copy(src_h.at[pl.ds(off, CH)], isx[k], isem[k])
        pltpu.async_copy(dst_h.at[pl.ds(off, CH)], idx[k], isem[k])

    def start_gather(c, k):
        off = base + c * CH
        pltpu.make_async_copy(src_h.at[pl.ds(off, CH)], isx[k], isem[k]).wait()
        pltpu.make_async_copy(dst_h.at[pl.ds(off, CH)], idx[k], isem[k]).wait()
        pltpu.async_copy(x_h.at[isx[k]], rows[k], gsem[k])

    def finish(c, k):
        pltpu.make_async_copy(x_h.at[isx[k]], rows[k], gsem[k]).wait()
        scatter(k)

    for k in range(3):
        start_idx(k, k)
        start_gather(k, k)

    def ring(j, carry):
        c0 = 3 * j
        finish(c0, 0)
        finish(c0 + 1, 1)
        drain(0)

        @pl.when(c0 + 3 < NCHUNK)
        def _():
            start_idx(c0 + 3, 0)

        finish(c0 + 2, 2)
        drain(1)

        @pl.when(c0 + 4 < NCHUNK)
        def _():
            start_idx(c0 + 4, 1)

        @pl.when(c0 + 3 < NCHUNK)
        def _():
            start_gather(c0 + 3, 0)

        drain(2)

        @pl.when(c0 + 5 < NCHUNK)
        def _():
            start_idx(c0 + 5, 2)

        @pl.when(c0 + 4 < NCHUNK)
        def _():
            start_gather(c0 + 4, 1)

        @pl.when(c0 + 5 < NCHUNK)
        def _():
            start_gather(c0 + 5, 2)

        return carry

    lax.fori_loop(0, NTRIPLE, ring, 0)
    finish(NCHUNK - 2, 0)
    finish(NCHUNK - 1, 1)
    drain(0)
    drain(1)


def _sc_agg1_body(x_h, src_h, dst_h, z128_h, z1_h, ones_h, aggp_h, cntp_h,
                  is0, is1, is2, id0, id1, id2, rows0, rows1, rows2,
                  onesv, cvec, i0, i1, i2, g0, g1, g2, s0, s1, s2,
                  agg_sh, cnt_sh):
    c = lax.axis_index("c")
    s = lax.axis_index("s")
    wid = s * NC + c
    nck = jnp.where(s == NS - 1, NLAST, NFULL)

    pltpu.sync_copy(z128_h.at[pl.ds(0, CH)], rows0)
    pltpu.sync_copy(ones_h, onesv)
    pltpu.sync_copy(z1_h.at[pl.ds(0, RINIT)], cvec)

    def zinit(i, carry):
        r = s * RINIT + i * CH
        pltpu.sync_copy(rows0, agg_sh.at[pl.ds(r, CH)])
        return carry

    lax.fori_loop(0, nck, zinit, 0)

    r0 = s * RINIT

    @pl.when(s < NS - 1)
    def _():
        pltpu.sync_copy(cvec, cnt_sh.at[pl.ds(r0, RINIT)])

    @pl.when(s == NS - 1)
    def _():
        pltpu.sync_copy(cvec.at[pl.ds(0, TAILN)], cnt_sh.at[pl.ds(TAIL0, TAILN)])

    plsc.subcore_barrier()

    rows = [rows0, rows1, rows2]
    isx = [is0, is1, is2]
    idx = [id0, id1, id2]
    isem = [i0, i1, i2]
    gsem = [g0, g1, g2]
    ssem = [s0, s1, s2]

    def scatter(k):
        pltpu.async_copy(rows[k], agg_sh.at[idx[k]], ssem[k], add=True)
        pltpu.async_copy(onesv, cnt_sh.at[idx[k]], ssem[k], add=True)

    def drain(k):
        pltpu.make_async_copy(rows[k], agg_sh.at[idx[k]], ssem[k]).wait()
        pltpu.make_async_copy(onesv, cnt_sh.at[idx[k]], ssem[k]).wait()

    _ring_edge_loop(x_h, src_h, dst_h, wid * EPW, rows, isx, idx, isem,
                    gsem, scatter, drain)
    plsc.subcore_barrier()

    def copyout(i, carry):
        r = s * RINIT + i * CH
        pltpu.sync_copy(agg_sh.at[pl.ds(r, CH)], rows0)
        pltpu.sync_copy(rows0, aggp_h.at[c, pl.ds(r, CH)])
        return carry

    lax.fori_loop(0, nck, copyout, 0)

    @pl.when(s < NS - 1)
    def _():
        pltpu.sync_copy(cnt_sh.at[pl.ds(r0, RINIT)], cvec)
        pltpu.sync_copy(cvec, cntp_h.at[pl.ds(c * N + r0, RINIT)])

    @pl.when(s == NS - 1)
    def _():
        pltpu.sync_copy(cnt_sh.at[pl.ds(TAIL0, TAILN)], cvec.at[pl.ds(0, TAILN)])
        pltpu.sync_copy(cvec.at[pl.ds(0, TAILN)],
                        cntp_h.at[pl.ds(c * N + TAIL0, TAILN)])


def _sc_agg2_body(x_h, src_h, dst_h, z128_h, aggp_h,
                  is0, is1, is2, id0, id1, id2, rows0, rows1, rows2,
                  i0, i1, i2, g0, g1, g2, s0, s1, s2, agg_sh):
    c = lax.axis_index("c")
    s = lax.axis_index("s")
    wid = s * NC + c
    nck = jnp.where(s == NS - 1, NLAST, NFULL)

    pltpu.sync_copy(z128_h.at[pl.ds(0, CH)], rows0)

    def zinit(i, carry):
        r = s * RINIT + i * CH
        pltpu.sync_copy(rows0, agg_sh.at[pl.ds(r, CH)])
        return carry

    lax.fori_loop(0, nck, zinit, 0)
    plsc.subcore_barrier()

    rows = [rows0, rows1, rows2]
    isx = [is0, is1, is2]
    idx = [id0, id1, id2]
    isem = [i0, i1, i2]
    gsem = [g0, g1, g2]
    ssem = [s0, s1, s2]

    def scatter(k):
        pltpu.async_copy(rows[k], agg_sh.at[idx[k]], ssem[k], add=True)

    def drain(k):
        pltpu.make_async_copy(rows[k], agg_sh.at[idx[k]], ssem[k]).wait()

    _ring_edge_loop(x_h, src_h, dst_h, wid * EPW, rows, isx, idx, isem,
                    gsem, scatter, drain)
    plsc.subcore_barrier()

    def copyout(i, carry):
        r = s * RINIT + i * CH
        pltpu.sync_copy(agg_sh.at[pl.ds(r, CH)], rows0)
        pltpu.sync_copy(rows0, aggp_h.at[c, pl.ds(r, CH)])
        return carry

    lax.fori_loop(0, nck, copyout, 0)


_sc_agg1 = pl.kernel(
    _sc_agg1_body,
    mesh=_MESH,
    out_type=[
        jax.ShapeDtypeStruct((NC, N, D_IN), jnp.float32),
        jax.ShapeDtypeStruct((NC * N,), jnp.float32),
    ],
    scratch_types=[
        pltpu.VMEM((CH,), jnp.int32),
        pltpu.VMEM((CH,), jnp.int32),
        pltpu.VMEM((CH,), jnp.int32),
        pltpu.VMEM((CH,), jnp.int32),
        pltpu.VMEM((CH,), jnp.int32),
        pltpu.VMEM((CH,), jnp.int32),
        pltpu.VMEM((CH, D_IN), jnp.float32),
        pltpu.VMEM((CH, D_IN), jnp.float32),
        pltpu.VMEM((CH, D_IN), jnp.float32),
        pltpu.VMEM((CH,), jnp.float32),
        pltpu.VMEM((RINIT,), jnp.float32),
        pltpu.SemaphoreType.DMA,
        pltpu.SemaphoreType.DMA,
        pltpu.SemaphoreType.DMA,
        pltpu.SemaphoreType.DMA,
        pltpu.SemaphoreType.DMA,
        pltpu.SemaphoreType.DMA,
        pltpu.SemaphoreType.DMA,
        pltpu.SemaphoreType.DMA,
        pltpu.SemaphoreType.DMA,
        pltpu.VMEM_SHARED((N, D_IN), jnp.float32),
        pltpu.VMEM_SHARED((N,), jnp.float32),
    ],
)

_sc_agg2 = pl.kernel(
    _sc_agg2_body,
    mesh=_MESH,
    out_type=jax.ShapeDtypeStruct((NC, N, D2G), jnp.float32),
    scratch_types=[
        pltpu.VMEM((CH,), jnp.int32),
        pltpu.VMEM((CH,), jnp.int32),
        pltpu.VMEM((CH,), jnp.int32),
        pltpu.VMEM((CH,), jnp.int32),
        pltpu.VMEM((CH,), jnp.int32),
        pltpu.VMEM((CH,), jnp.int32),
        pltpu.VMEM((CH, D2G), jnp.float32),
        pltpu.VMEM((CH, D2G), jnp.float32),
        pltpu.VMEM((CH, D2G), jnp.float32),
        pltpu.SemaphoreType.DMA,
        pltpu.SemaphoreType.DMA,
        pltpu.SemaphoreType.DMA,
        pltpu.SemaphoreType.DMA,
        pltpu.SemaphoreType.DMA,
        pltpu.SemaphoreType.DMA,
        pltpu.SemaphoreType.DMA,
        pltpu.SemaphoreType.DMA,
        pltpu.SemaphoreType.DMA,
        pltpu.VMEM_SHARED((N, D2G), jnp.float32),
    ],
)


def _dense1_body(x_r, aggp_r, cnt_r, w1l_r, w1r_r, b1_r, scale_r, beta_r,
                 w2l_r, w2r_r, b2_r, y2_r, hr_r):
    agg = aggp_r[0] + aggp_r[1]
    mean = agg / jnp.maximum(cnt_r[...], 1.0)
    h = (jnp.dot(mean, w1l_r[...], preferred_element_type=jnp.float32)
         + jnp.dot(x_r[...], w1r_r[...], preferred_element_type=jnp.float32)
         + b1_r[...])
    h = h * scale_r[...] + beta_r[...]
    h = jnp.maximum(h, 0.0)
    y2_r[...] = jnp.dot(h, w2l_r[...], preferred_element_type=jnp.float32)
    hr_r[...] = (jnp.dot(h, w2r_r[...], preferred_element_type=jnp.float32)
                 + b2_r[...])


def _dense2_body(aggp_r, cnt_r, hr_r, out_r):
    agg = aggp_r[0, :, :D2P] + aggp_r[1, :, :D2P]
    logits = agg / jnp.maximum(cnt_r[...], 1.0) + hr_r[...]
    col = lax.broadcasted_iota(jnp.int32, logits.shape, 1)
    valid = col < D_OUT
    neg = jnp.float32(-3.0e38)
    masked = jnp.where(valid, logits, neg)
    m = jnp.max(masked, axis=-1, keepdims=True)
    e = jnp.where(valid, jnp.exp(logits - m), 0.0)
    lse = jnp.log(jnp.sum(e, axis=-1, keepdims=True))
    out_r[...] = (logits - m - lse)[:, :D_OUT]


_BR = 2000


def _dense1_call(x, aggp, cnt, w1l, w1r, b1, scale, beta, w2l, w2r, b2):
    grid = (N // _BR,)
    row = lambda i: (i, 0)
    fixed2 = lambda i: (0, 0)
    return pl.pallas_call(
        _dense1_body,
        grid=grid,
        in_specs=[
            pl.BlockSpec((_BR, D_IN), row),
            pl.BlockSpec((NC, _BR, D_IN), lambda i: (0, i, 0)),
            pl.BlockSpec((_BR, 1), row),
            pl.BlockSpec((D_IN, D_H), fixed2),
            pl.BlockSpec((D_IN, D_H), fixed2),
            pl.BlockSpec((1, D_H), fixed2),
            pl.BlockSpec((1, D_H), fixed2),
            pl.BlockSpec((1, D_H), fixed2),
            pl.BlockSpec((D_H, D2G), fixed2),
            pl.BlockSpec((D_H, D2P), fixed2),
            pl.BlockSpec((1, D2P), fixed2),
        ],
        out_specs=[
            pl.BlockSpec((_BR, D2G), row),
            pl.BlockSpec((_BR, D2P), row),
        ],
        out_shape=[
            jax.ShapeDtypeStruct((N, D2G), jnp.float32),
            jax.ShapeDtypeStruct((N, D2P), jnp.float32),
        ],
    )(x, aggp, cnt, w1l, w1r, b1, scale, beta, w2l, w2r, b2)


def _dense2_call(aggp, cnt, hr):
    grid = (N // _BR,)
    row = lambda i: (i, 0)
    return pl.pallas_call(
        _dense2_body,
        grid=grid,
        in_specs=[
            pl.BlockSpec((NC, _BR, D2G), lambda i: (0, i, 0)),
            pl.BlockSpec((_BR, 1), row),
            pl.BlockSpec((_BR, D2P), row),
        ],
        out_specs=pl.BlockSpec((_BR, D_OUT), row),
        out_shape=jax.ShapeDtypeStruct((N, D_OUT), jnp.float32),
    )(aggp, cnt, hr)


def kernel(x, edge_index, W1l, W1r, b1, gamma1, beta1, W2l, W2r, b2):
    src = edge_index[0]
    dst = edge_index[1]
    z128 = jnp.zeros((N, D_IN), jnp.float32)
    z1 = jnp.zeros((N,), jnp.float32)
    ones1 = jnp.ones((CH,), jnp.float32)

    aggp, cntp = _sc_agg1(x, src, dst, z128, z1, ones1)
    cnt2 = cntp.reshape(NC, N)
    cnt = (cnt2[0] + cnt2[1]).reshape(N, 1)

    scale = (gamma1 / jnp.sqrt(1.0 + BN_EPS)).reshape(1, D_H)
    w2l_pad = jnp.zeros((D_H, D2G), jnp.float32).at[:, :D_OUT].set(W2l)
    w2r_pad = jnp.zeros((D_H, D2P), jnp.float32).at[:, :D_OUT].set(W2r)
    b2_pad = jnp.zeros((1, D2P), jnp.float32).at[0, :D_OUT].set(b2)

    y2, hr = _dense1_call(x, aggp, cnt, W1l, W1r, b1.reshape(1, D_H),
                          scale, beta1.reshape(1, D_H), w2l_pad, w2r_pad,
                          b2_pad)

    agg2p = _sc_agg2(y2, src, dst, z128)

    return _dense2_call(agg2p, cnt, hr)


# speedup vs baseline: 1.2229x; 1.2229x over previous
import jax
import jax.numpy as jnp
from jax import lax
from jax.experimental import pallas as pl
from jax.experimental.pallas import tpu as pltpu
from jax.experimental.pallas import tpu_sc as plsc

N = 10000
E = 320000
D_IN = 128
D_H = 128
D_OUT = 40
D2P = 64
D2G = 128
BN_EPS = 1e-5

NC = 2
NS = 16
NW = NC * NS
EPW = E // NW
CH = 80
NCHUNK = EPW // CH
NQUAD = 31
RINIT = 640
NFULL = 8
NLAST = 5
TAIL0 = 15 * RINIT
TAILN = N - TAIL0

_MESH = plsc.VectorSubcoreMesh(core_axis_name="c", subcore_axis_name="s")


def _ring_edge_loop(x_h, src_h, dst_h, base, rows, isx, idx, isem, gsem,
                    scatter, drain):
    def start_idx(c, k):
        off = base + c * CH
        pltpu.async_copy(src_h.at[pl.ds(off, CH)], isx[k], isem[k])
        pltpu.async_copy(dst_h.at[pl.ds(off, CH)], idx[k], isem[k])

    def start_gather(c, k):
        off = base + c * CH
        pltpu.make_async_copy(src_h.at[pl.ds(off, CH)], isx[k], isem[k]).wait()
        pltpu.make_async_copy(dst_h.at[pl.ds(off, CH)], idx[k], isem[k]).wait()
        pltpu.async_copy(x_h.at[isx[k]], rows[k], gsem[k])

    def finish(c, k):
        pltpu.make_async_copy(x_h.at[isx[k]], rows[k], gsem[k]).wait()
        scatter(k)

    for k in range(4):
        start_idx(k, k)
        start_gather(k, k)

    def ring(j, carry):
        c0 = 4 * j
        finish(c0, 0)
        drain(0)

        @pl.when(c0 + 4 < NCHUNK)
        def _():
            start_idx(c0 + 4, 0)

        finish(c0 + 1, 1)

        @pl.when(c0 + 4 < NCHUNK)
        def _():
            start_gather(c0 + 4, 0)

        drain(1)

        @pl.when(c0 + 5 < NCHUNK)
        def _():
            start_idx(c0 + 5, 1)

        finish(c0 + 2, 2)

        @pl.when(c0 + 5 < NCHUNK)
        def _():
            start_gather(c0 + 5, 1)

        drain(2)

        @pl.when(c0 + 6 < NCHUNK)
        def _():
            start_idx(c0 + 6, 2)

        finish(c0 + 3, 3)

        @pl.when(c0 + 6 < NCHUNK)
        def _():
            start_gather(c0 + 6, 2)

        drain(3)

        @pl.when(c0 + 7 < NCHUNK)
        def _():
            start_idx(c0 + 7, 3)
            start_gather(c0 + 7, 3)

        return carry

    lax.fori_loop(0, NQUAD, ring, 0)
    finish(NCHUNK - 1, 0)
    drain(0)


def _sc_agg1_body(x_h, src_h, dst_h, z128_h, z1_h, ones_h, aggp_h, cntp_h,
                  is0, is1, is2, is3, id0, id1, id2, id3,
                  rows0, rows1, rows2, rows3,
                  onesv, cvec, i0, i1, i2, i3, g0, g1, g2, g3,
                  s0, s1, s2, s3, agg_sh, cnt_sh):
    c = lax.axis_index("c")
    s = lax.axis_index("s")
    wid = s * NC + c
    nck = jnp.where(s == NS - 1, NLAST, NFULL)

    pltpu.sync_copy(z128_h.at[pl.ds(0, CH)], rows0)
    pltpu.sync_copy(ones_h, onesv)
    pltpu.sync_copy(z1_h.at[pl.ds(0, RINIT)], cvec)

    def zinit(i, carry):
        r = s * RINIT + i * CH
        pltpu.sync_copy(rows0, agg_sh.at[pl.ds(r, CH)])
        return carry

    lax.fori_loop(0, nck, zinit, 0)

    r0 = s * RINIT

    @pl.when(s < NS - 1)
    def _():
        pltpu.sync_copy(cvec, cnt_sh.at[pl.ds(r0, RINIT)])

    @pl.when(s == NS - 1)
    def _():
        pltpu.sync_copy(cvec.at[pl.ds(0, TAILN)], cnt_sh.at[pl.ds(TAIL0, TAILN)])

    plsc.subcore_barrier()

    rows = [rows0, rows1, rows2, rows3]
    isx = [is0, is1, is2, is3]
    idx = [id0, id1, id2, id3]
    isem = [i0, i1, i2, i3]
    gsem = [g0, g1, g2, g3]
    ssem = [s0, s1, s2, s3]

    def scatter(k):
        pltpu.async_copy(rows[k], agg_sh.at[idx[k]], ssem[k], add=True)
        pltpu.async_copy(onesv, cnt_sh.at[idx[k]], ssem[k], add=True)

    def drain(k):
        pltpu.make_async_copy(rows[k], agg_sh.at[idx[k]], ssem[k]).wait()
        pltpu.make_async_copy(onesv, cnt_sh.at[idx[k]], ssem[k]).wait()

    _ring_edge_loop(x_h, src_h, dst_h, wid * EPW, rows, isx, idx, isem,
                    gsem, scatter, drain)
    plsc.subcore_barrier()

    def copyout(i, carry):
        r = s * RINIT + i * CH
        pltpu.sync_copy(agg_sh.at[pl.ds(r, CH)], rows0)
        pltpu.sync_copy(rows0, aggp_h.at[c, pl.ds(r, CH)])
        return carry

    lax.fori_loop(0, nck, copyout, 0)

    @pl.when(s < NS - 1)
    def _():
        pltpu.sync_copy(cnt_sh.at[pl.ds(r0, RINIT)], cvec)
        pltpu.sync_copy(cvec, cntp_h.at[pl.ds(c * N + r0, RINIT)])

    @pl.when(s == NS - 1)
    def _():
        pltpu.sync_copy(cnt_sh.at[pl.ds(TAIL0, TAILN)], cvec.at[pl.ds(0, TAILN)])
        pltpu.sync_copy(cvec.at[pl.ds(0, TAILN)],
                        cntp_h.at[pl.ds(c * N + TAIL0, TAILN)])


def _sc_agg2_body(x_h, src_h, dst_h, z128_h, aggp_h,
                  is0, is1, is2, is3, id0, id1, id2, id3,
                  rows0, rows1, rows2, rows3,
                  i0, i1, i2, i3, g0, g1, g2, g3, s0, s1, s2, s3, agg_sh):
    c = lax.axis_index("c")
    s = lax.axis_index("s")
    wid = s * NC + c
    nck = jnp.where(s == NS - 1, NLAST, NFULL)

    pltpu.sync_copy(z128_h.at[pl.ds(0, CH)], rows0)

    def zinit(i, carry):
        r = s * RINIT + i * CH
        pltpu.sync_copy(rows0, agg_sh.at[pl.ds(r, CH)])
        return carry

    lax.fori_loop(0, nck, zinit, 0)
    plsc.subcore_barrier()

    rows = [rows0, rows1, rows2, rows3]
    isx = [is0, is1, is2, is3]
    idx = [id0, id1, id2, id3]
    isem = [i0, i1, i2, i3]
    gsem = [g0, g1, g2, g3]
    ssem = [s0, s1, s2, s3]

    def scatter(k):
        pltpu.async_copy(rows[k], agg_sh.at[idx[k]], ssem[k], add=True)

    def drain(k):
        pltpu.make_async_copy(rows[k], agg_sh.at[idx[k]], ssem[k]).wait()

    _ring_edge_loop(x_h, src_h, dst_h, wid * EPW, rows, isx, idx, isem,
                    gsem, scatter, drain)
    plsc.subcore_barrier()

    def copyout(i, carry):
        r = s * RINIT + i * CH
        pltpu.sync_copy(agg_sh.at[pl.ds(r, CH)], rows0)
        pltpu.sync_copy(rows0, aggp_h.at[c, pl.ds(r, CH)])
        return carry

    lax.fori_loop(0, nck, copyout, 0)


_sc_agg1 = pl.kernel(
    _sc_agg1_body,
    mesh=_MESH,
    out_type=[
        jax.ShapeDtypeStruct((NC, N, D_IN), jnp.float32),
        jax.ShapeDtypeStruct((NC * N,), jnp.float32),
    ],
    scratch_types=(
        [pltpu.VMEM((CH,), jnp.int32)] * 8
        + [pltpu.VMEM((CH, D_IN), jnp.float32)] * 4
        + [pltpu.VMEM((CH,), jnp.float32),
           pltpu.VMEM((RINIT,), jnp.float32)]
        + [pltpu.SemaphoreType.DMA] * 12
        + [pltpu.VMEM_SHARED((N, D_IN), jnp.float32),
           pltpu.VMEM_SHARED((N,), jnp.float32)]
    ),
)

_sc_agg2 = pl.kernel(
    _sc_agg2_body,
    mesh=_MESH,
    out_type=jax.ShapeDtypeStruct((NC, N, D2G), jnp.float32),
    scratch_types=(
        [pltpu.VMEM((CH,), jnp.int32)] * 8
        + [pltpu.VMEM((CH, D2G), jnp.float32)] * 4
        + [pltpu.SemaphoreType.DMA] * 12
        + [pltpu.VMEM_SHARED((N, D2G), jnp.float32)]
    ),
)


def _dense1_body(x_r, aggp_r, cnt_r, w1l_r, w1r_r, b1_r, scale_r, beta_r,
                 w2l_r, w2r_r, b2_r, y2_r, hr_r):
    agg = aggp_r[0] + aggp_r[1]
    mean = agg / jnp.maximum(cnt_r[...], 1.0)
    h = (jnp.dot(mean, w1l_r[...], preferred_element_type=jnp.float32)
         + jnp.dot(x_r[...], w1r_r[...], preferred_element_type=jnp.float32)
         + b1_r[...])
    h = h * scale_r[...] + beta_r[...]
    h = jnp.maximum(h, 0.0)
    y2_r[...] = jnp.dot(h, w2l_r[...], preferred_element_type=jnp.float32)
    hr_r[...] = (jnp.dot(h, w2r_r[...], preferred_element_type=jnp.float32)
                 + b2_r[...])


def _dense2_body(aggp_r, cnt_r, hr_r, out_r):
    agg = aggp_r[0, :, :D2P] + aggp_r[1, :, :D2P]
    logits = agg / jnp.maximum(cnt_r[...], 1.0) + hr_r[...]
    col = lax.broadcasted_iota(jnp.int32, logits.shape, 1)
    valid = col < D_OUT
    neg = jnp.float32(-3.0e38)
    masked = jnp.where(valid, logits, neg)
    m = jnp.max(masked, axis=-1, keepdims=True)
    e = jnp.where(valid, jnp.exp(logits - m), 0.0)
    lse = jnp.log(jnp.sum(e, axis=-1, keepdims=True))
    out_r[...] = (logits - m - lse)[:, :D_OUT]


_BR = 2000


def _dense1_call(x, aggp, cnt, w1l, w1r, b1, scale, beta, w2l, w2r, b2):
    grid = (N // _BR,)
    row = lambda i: (i, 0)
    fixed2 = lambda i: (0, 0)
    return pl.pallas_call(
        _dense1_body,
        grid=grid,
        in_specs=[
            pl.BlockSpec((_BR, D_IN), row),
            pl.BlockSpec((NC, _BR, D_IN), lambda i: (0, i, 0)),
            pl.BlockSpec((_BR, 1), row),
            pl.BlockSpec((D_IN, D_H), fixed2),
            pl.BlockSpec((D_IN, D_H), fixed2),
            pl.BlockSpec((1, D_H), fixed2),
            pl.BlockSpec((1, D_H), fixed2),
            pl.BlockSpec((1, D_H), fixed2),
            pl.BlockSpec((D_H, D2G), fixed2),
            pl.BlockSpec((D_H, D2P), fixed2),
            pl.BlockSpec((1, D2P), fixed2),
        ],
        out_specs=[
            pl.BlockSpec((_BR, D2G), row),
            pl.BlockSpec((_BR, D2P), row),
        ],
        out_shape=[
            jax.ShapeDtypeStruct((N, D2G), jnp.float32),
            jax.ShapeDtypeStruct((N, D2P), jnp.float32),
        ],
    )(x, aggp, cnt, w1l, w1r, b1, scale, beta, w2l, w2r, b2)


def _dense2_call(aggp, cnt, hr):
    grid = (N // _BR,)
    row = lambda i: (i, 0)
    return pl.pallas_call(
        _dense2_body,
        grid=grid,
        in_specs=[
            pl.BlockSpec((NC, _BR, D2G), lambda i: (0, i, 0)),
            pl.BlockSpec((_BR, 1), row),
            pl.BlockSpec((_BR, D2P), row),
        ],
        out_specs=pl.BlockSpec((_BR, D_OUT), row),
        out_shape=jax.ShapeDtypeStruct((N, D_OUT), jnp.float32),
    )(aggp, cnt, hr)


def kernel(x, edge_index, W1l, W1r, b1, gamma1, beta1, W2l, W2r, b2):
    src = edge_index[0]
    dst = edge_index[1]
    z128 = jnp.zeros((N, D_IN), jnp.float32)
    z1 = jnp.zeros((N,), jnp.float32)
    ones1 = jnp.ones((CH,), jnp.float32)

    aggp, cntp = _sc_agg1(x, src, dst, z128, z1, ones1)
    cnt2 = cntp.reshape(NC, N)
    cnt = (cnt2[0] + cnt2[1]).reshape(N, 1)

    scale = (gamma1 / jnp.sqrt(1.0 + BN_EPS)).reshape(1, D_H)
    w2l_pad = jnp.zeros((D_H, D2G), jnp.float32).at[:, :D_OUT].set(W2l)
    w2r_pad = jnp.zeros((D_H, D2P), jnp.float32).at[:, :D_OUT].set(W2r)
    b2_pad = jnp.zeros((1, D2P), jnp.float32).at[0, :D_OUT].set(b2)

    y2, hr = _dense1_call(x, aggp, cnt, W1l, W1r, b1.reshape(1, D_H),
                          scale, beta1.reshape(1, D_H), w2l_pad, w2r_pad,
                          b2_pad)

    agg2p = _sc_agg2(y2, src, dst, z128)

    return _dense2_call(agg2p, cnt, hr)
